# baseline (speedup 1.0000x reference)
import functools

import jax
import jax.numpy as jnp
from jax import lax
from jax.experimental import pallas as pl
from jax.experimental.pallas import tpu as pltpu
from jax.experimental.pallas import tpu_sc as plsc

B = 32
IN_DIM = 8192
HID = 256
K = 4096

NVEC = IN_DIM // 16


def _score_body(x_ref, W1_ref, b1_ref, ln1_g_ref, ln1_b_ref, W2_ref, b2_ref,
                W3_ref, b3_ref, Wg1_ref, bg1_ref, Wg2_ref, bg2_ref,
                key_out_ref):
    x = x_ref[...]
    h = jnp.dot(x, W1_ref[...], preferred_element_type=jnp.float32) + b1_ref[...]
    m = jnp.mean(h, axis=-1, keepdims=True)
    v = jnp.mean((h - m) ** 2, axis=-1, keepdims=True)
    h = (h - m) * jax.lax.rsqrt(v + 1e-5) * ln1_g_ref[...] + ln1_b_ref[...]
    h = jnp.maximum(h, 0.0)
    h = jnp.maximum(jnp.dot(h, W2_ref[...], preferred_element_type=jnp.float32) + b2_ref[...], 0.0)
    imp = jnp.dot(h, W3_ref[...], preferred_element_type=jnp.float32) + b3_ref[...]

    g = jnp.maximum(jnp.dot(x, Wg1_ref[...], preferred_element_type=jnp.float32) + bg1_ref[...], 0.0)
    gz = jnp.dot(g, Wg2_ref[...], preferred_element_type=jnp.float32) + bg2_ref[...]
    gates = 1.0 / (1.0 + jnp.exp(-gz))

    sel = imp * gates
    bits = jax.lax.bitcast_convert_type(sel, jnp.int32)
    key_out_ref[...] = jnp.where(bits >= 0, bits, bits ^ jnp.int32(0x7FFFFFFF))


def _thresh_kernel_body(key_hbm, th_hbm, key_v, th_v):
    wid = lax.axis_index("s") * 2 + lax.axis_index("c")
    pltpu.sync_copy(key_hbm.at[wid], key_v)

    def count_ge(cand):
        def body(i, acc):
            v = key_v[pl.ds(i * 16, 16)]
            return acc + jnp.where(v >= cand, jnp.int32(1), jnp.int32(0))

        acc = lax.fori_loop(0, NVEC, body, jnp.zeros((16,), jnp.int32))
        total = jnp.int32(0)
        for j in range(16):
            total = total + acc[j]
        return total

    cnt0 = count_ge(jnp.int32(0))
    take0 = cnt0 >= K
    thresh = jnp.where(take0, jnp.int32(0), jnp.int32(-2147483648))

    def step(i, th):
        cand = th | (jnp.int32(1) << (jnp.int32(30) - i))
        c = count_ge(cand)
        return jnp.where(c >= K, cand, th)

    thresh = lax.fori_loop(0, 31, step, thresh)

    th_v[...] = jnp.full((16,), jnp.int32(0)) + thresh
    pltpu.sync_copy(th_v, th_hbm.at[wid])


@functools.partial(
    pl.kernel,
    mesh=plsc.VectorSubcoreMesh(core_axis_name="c", subcore_axis_name="s"),
    out_type=jax.ShapeDtypeStruct((B, 16), jnp.int32),
    scratch_types=[
        pltpu.VMEM((IN_DIM,), jnp.int32),
        pltpu.VMEM((16,), jnp.int32),
    ],
)
def _thresh_sc(key_hbm, th_hbm, key_v, th_v):
    _thresh_kernel_body(key_hbm, th_hbm, key_v, th_v)


def _transform_body(x_ref, key_ref, th_ref, Wt_ref, bt_ref, lnt_g_ref,
                    lnt_b_ref, t_out_ref, p_out_ref):
    mask = (key_ref[...] >= th_ref[...][:, 0:1]).astype(jnp.float32)
    p_out_ref[...] = mask
    t = jnp.dot(x_ref[...] * mask, Wt_ref[...],
                preferred_element_type=jnp.float32) + bt_ref[...]
    mt = jnp.mean(t, axis=-1, keepdims=True)
    vt = jnp.mean((t - mt) ** 2, axis=-1, keepdims=True)
    t = (t - mt) * jax.lax.rsqrt(vt + 1e-5) * lnt_g_ref[...] + lnt_b_ref[...]
    t_out_ref[...] = jnp.maximum(t, 0.0)


@jax.jit
def _run(x, W1, b1, ln1_g, ln1_b, W2, b2, W3, b3, Wg1, bg1, Wg2, bg2,
         Wt, bt, lnt_g, lnt_b):
    keys = pl.pallas_call(
        _score_body,
        out_shape=jax.ShapeDtypeStruct((B, IN_DIM), jnp.int32),
        compiler_params=pltpu.CompilerParams(vmem_limit_bytes=100 * 1024 * 1024),
    )(x, W1, b1, ln1_g, ln1_b, W2, b2, W3, b3, Wg1, bg1, Wg2, bg2)

    th = _thresh_sc(keys)

    transformed, mask = pl.pallas_call(
        _transform_body,
        out_shape=(
            jax.ShapeDtypeStruct((B, HID), jnp.float32),
            jax.ShapeDtypeStruct((B, IN_DIM), jnp.float32),
        ),
        compiler_params=pltpu.CompilerParams(vmem_limit_bytes=100 * 1024 * 1024),
    )(x, keys, th, Wt, bt, lnt_g, lnt_b)
    return transformed, mask


def kernel(x, W1, b1, ln1_g, ln1_b, W2, b2, W3, b3, Wg1, bg1, Wg2, bg2, Wt, bt, lnt_g, lnt_b):
    return _run(
        x, W1, b1.reshape(1, -1), ln1_g.reshape(1, -1), ln1_b.reshape(1, -1),
        W2, b2.reshape(1, -1), W3, b3.reshape(1, -1),
        Wg1, bg1.reshape(1, -1), Wg2, bg2.reshape(1, -1),
        Wt, bt.reshape(1, -1), lnt_g.reshape(1, -1), lnt_b.reshape(1, -1),
    )


# speedup vs baseline: 1.7237x; 1.7237x over previous
import functools

import jax
import jax.numpy as jnp
from jax import lax
from jax.experimental import pallas as pl
from jax.experimental.pallas import tpu as pltpu
from jax.experimental.pallas import tpu_sc as plsc

B = 32
IN_DIM = 8192
HID = 256
K = 4096

NVEC = IN_DIM // 16


def _score_body(x_ref, W1_ref, b1_ref, ln1_g_ref, ln1_b_ref, W2_ref, b2_ref,
                W3_ref, b3_ref, Wg1_ref, bg1_ref, Wg2_ref, bg2_ref,
                key_out_ref):
    x = x_ref[...]
    h = jnp.dot(x, W1_ref[...], preferred_element_type=jnp.float32) + b1_ref[...]
    m = jnp.mean(h, axis=-1, keepdims=True)
    v = jnp.mean((h - m) ** 2, axis=-1, keepdims=True)
    h = (h - m) * jax.lax.rsqrt(v + 1e-5) * ln1_g_ref[...] + ln1_b_ref[...]
    h = jnp.maximum(h, 0.0)
    h = jnp.maximum(jnp.dot(h, W2_ref[...], preferred_element_type=jnp.float32) + b2_ref[...], 0.0)
    imp = jnp.dot(h, W3_ref[...], preferred_element_type=jnp.float32) + b3_ref[...]

    g = jnp.maximum(jnp.dot(x, Wg1_ref[...], preferred_element_type=jnp.float32) + bg1_ref[...], 0.0)
    gz = jnp.dot(g, Wg2_ref[...], preferred_element_type=jnp.float32) + bg2_ref[...]
    gates = 1.0 / (1.0 + jnp.exp(-gz))

    sel = imp * gates
    bits = jax.lax.bitcast_convert_type(sel, jnp.int32)
    key_out_ref[...] = jnp.where(bits >= 0, bits, bits ^ jnp.int32(0x7FFFFFFF))


def _thresh_kernel_body(key_hbm, th_hbm, key_v, th_v):
    wid = lax.axis_index("s") * 2 + lax.axis_index("c")
    pltpu.sync_copy(key_hbm.at[wid], key_v)

    UNROLL = 4

    def count3_ge(ca, cb, cc):
        one = jnp.int32(1)
        zero = jnp.int32(0)

        def body(i, accs):
            aa, ab, ac = accs
            for u in range(UNROLL):
                v = key_v[pl.ds((i * UNROLL + u) * 16, 16)]
                aa = aa + jnp.where(v >= ca, one, zero)
                ab = ab + jnp.where(v >= cb, one, zero)
                ac = ac + jnp.where(v >= cc, one, zero)
            return aa, ab, ac

        z = jnp.zeros((16,), jnp.int32)
        aa, ab, ac = lax.fori_loop(0, NVEC // UNROLL, body, (z, z, z))
        ta = tb = tc = jnp.int32(0)
        for j in range(16):
            ta = ta + aa[j]
            tb = tb + ab[j]
            tc = tc + ac[j]
        return ta, tb, tc

    thresh = jnp.int32(-2147483648)
    for p in range(16):
        b_hi = 31 - 2 * p
        bit_lo = jnp.int32(1) << (b_hi - 1)
        if p == 0:
            ca = jnp.int32(0)
            base = thresh
        else:
            ca = thresh | (jnp.int32(1) << b_hi)
            base = thresh
        cb = base | bit_lo
        cc = ca | bit_lo
        c_a, c_b, c_c = count3_ge(ca, cb, cc)
        thresh = jnp.where(
            c_a >= K,
            jnp.where(c_c >= K, cc, ca),
            jnp.where(c_b >= K, cb, base),
        )

    th_v[...] = jnp.full((16,), jnp.int32(0)) + thresh
    pltpu.sync_copy(th_v, th_hbm.at[wid])


@functools.partial(
    pl.kernel,
    mesh=plsc.VectorSubcoreMesh(core_axis_name="c", subcore_axis_name="s"),
    out_type=jax.ShapeDtypeStruct((B, 16), jnp.int32),
    scratch_types=[
        pltpu.VMEM((IN_DIM,), jnp.int32),
        pltpu.VMEM((16,), jnp.int32),
    ],
)
def _thresh_sc(key_hbm, th_hbm, key_v, th_v):
    _thresh_kernel_body(key_hbm, th_hbm, key_v, th_v)


def _transform_body(x_ref, key_ref, th_ref, Wt_ref, bt_ref, lnt_g_ref,
                    lnt_b_ref, t_out_ref, p_out_ref):
    mask = (key_ref[...] >= th_ref[...][:, 0:1]).astype(jnp.float32)
    p_out_ref[...] = mask
    t = jnp.dot(x_ref[...] * mask, Wt_ref[...],
                preferred_element_type=jnp.float32) + bt_ref[...]
    mt = jnp.mean(t, axis=-1, keepdims=True)
    vt = jnp.mean((t - mt) ** 2, axis=-1, keepdims=True)
    t = (t - mt) * jax.lax.rsqrt(vt + 1e-5) * lnt_g_ref[...] + lnt_b_ref[...]
    t_out_ref[...] = jnp.maximum(t, 0.0)


@jax.jit
def _run(x, W1, b1, ln1_g, ln1_b, W2, b2, W3, b3, Wg1, bg1, Wg2, bg2,
         Wt, bt, lnt_g, lnt_b):
    keys = pl.pallas_call(
        _score_body,
        out_shape=jax.ShapeDtypeStruct((B, IN_DIM), jnp.int32),
        compiler_params=pltpu.CompilerParams(vmem_limit_bytes=100 * 1024 * 1024),
    )(x, W1, b1, ln1_g, ln1_b, W2, b2, W3, b3, Wg1, bg1, Wg2, bg2)

    th = _thresh_sc(keys)

    transformed, mask = pl.pallas_call(
        _transform_body,
        out_shape=(
            jax.ShapeDtypeStruct((B, HID), jnp.float32),
            jax.ShapeDtypeStruct((B, IN_DIM), jnp.float32),
        ),
        compiler_params=pltpu.CompilerParams(vmem_limit_bytes=100 * 1024 * 1024),
    )(x, keys, th, Wt, bt, lnt_g, lnt_b)
    return transformed, mask


def kernel(x, W1, b1, ln1_g, ln1_b, W2, b2, W3, b3, Wg1, bg1, Wg2, bg2, Wt, bt, lnt_g, lnt_b):
    return _run(
        x, W1, b1.reshape(1, -1), ln1_g.reshape(1, -1), ln1_b.reshape(1, -1),
        W2, b2.reshape(1, -1), W3, b3.reshape(1, -1),
        Wg1, bg1.reshape(1, -1), Wg2, bg2.reshape(1, -1),
        Wt, bt.reshape(1, -1), lnt_g.reshape(1, -1), lnt_b.reshape(1, -1),
    )


# speedup vs baseline: 1.7365x; 1.0074x over previous
import functools

import jax
import jax.numpy as jnp
from jax import lax
from jax.experimental import pallas as pl
from jax.experimental.pallas import tpu as pltpu
from jax.experimental.pallas import tpu_sc as plsc

B = 32
IN_DIM = 8192
HID = 256
K = 4096

NVEC = IN_DIM // 16


def _score_body(x_ref, W1_ref, b1_ref, ln1_g_ref, ln1_b_ref, W2_ref, b2_ref,
                W3_ref, b3_ref, Wg1_ref, bg1_ref, Wg2_ref, bg2_ref,
                key_out_ref):
    x = x_ref[...]
    h = jnp.dot(x, W1_ref[...], preferred_element_type=jnp.float32) + b1_ref[...]
    m = jnp.mean(h, axis=-1, keepdims=True)
    v = jnp.mean((h - m) ** 2, axis=-1, keepdims=True)
    h = (h - m) * jax.lax.rsqrt(v + 1e-5) * ln1_g_ref[...] + ln1_b_ref[...]
    h = jnp.maximum(h, 0.0)
    h = jnp.maximum(jnp.dot(h, W2_ref[...], preferred_element_type=jnp.float32) + b2_ref[...], 0.0)
    imp = jnp.dot(h, W3_ref[...], preferred_element_type=jnp.float32) + b3_ref[...]

    g = jnp.maximum(jnp.dot(x, Wg1_ref[...], preferred_element_type=jnp.float32) + bg1_ref[...], 0.0)
    gz = jnp.dot(g, Wg2_ref[...], preferred_element_type=jnp.float32) + bg2_ref[...]
    gates = 1.0 / (1.0 + jnp.exp(-gz))

    sel = imp * gates
    bits = jax.lax.bitcast_convert_type(sel, jnp.int32)
    key_out_ref[...] = jnp.where(bits >= 0, bits, bits ^ jnp.int32(0x7FFFFFFF))


def _thresh_kernel_body(key_hbm, th_hbm, key_v, th_v):
    wid = lax.axis_index("s") * 2 + lax.axis_index("c")
    pltpu.sync_copy(key_hbm.at[wid], key_v)

    UNROLL = 2

    def count3_ge(ca, cb, cc):
        one = jnp.int32(1)
        zero = jnp.int32(0)

        def body(i, accs):
            aa, ab, ac = accs
            for u in range(UNROLL):
                v = key_v[pl.ds((i * UNROLL + u) * 16, 16)]
                aa = aa + jnp.where(v >= ca, one, zero)
                ab = ab + jnp.where(v >= cb, one, zero)
                ac = ac + jnp.where(v >= cc, one, zero)
            return aa, ab, ac

        z = jnp.zeros((16,), jnp.int32)
        aa, ab, ac = lax.fori_loop(0, NVEC // UNROLL, body, (z, z, z))
        ta = tb = tc = jnp.int32(0)
        for j in range(16):
            ta = ta + aa[j]
            tb = tb + ab[j]
            tc = tc + ac[j]
        return ta, tb, tc

    thresh = jnp.int32(-2147483648)
    for p in range(16):
        b_hi = 31 - 2 * p
        bit_lo = jnp.int32(1) << (b_hi - 1)
        if p == 0:
            ca = jnp.int32(0)
            base = thresh
        else:
            ca = thresh | (jnp.int32(1) << b_hi)
            base = thresh
        cb = base | bit_lo
        cc = ca | bit_lo
        c_a, c_b, c_c = count3_ge(ca, cb, cc)
        thresh = jnp.where(
            c_a >= K,
            jnp.where(c_c >= K, cc, ca),
            jnp.where(c_b >= K, cb, base),
        )

    th_v[...] = jnp.full((16,), jnp.int32(0)) + thresh
    pltpu.sync_copy(th_v, th_hbm.at[wid])


@functools.cache
def _thresh_sc_build():
    return functools.partial(
        pl.kernel,
        mesh=plsc.VectorSubcoreMesh(core_axis_name="c", subcore_axis_name="s"),
        out_type=jax.ShapeDtypeStruct((B, 16), jnp.int32),
        scratch_types=[
            pltpu.VMEM((IN_DIM,), jnp.int32),
            pltpu.VMEM((16,), jnp.int32),
        ],
    )(_thresh_kernel_body)


def _transform_body(x_ref, key_ref, th_ref, Wt_ref, bt_ref, lnt_g_ref,
                    lnt_b_ref, t_out_ref, p_out_ref):
    mask = (key_ref[...] >= th_ref[...][:, 0:1]).astype(jnp.float32)
    p_out_ref[...] = mask
    t = jnp.dot(x_ref[...] * mask, Wt_ref[...],
                preferred_element_type=jnp.float32) + bt_ref[...]
    mt = jnp.mean(t, axis=-1, keepdims=True)
    vt = jnp.mean((t - mt) ** 2, axis=-1, keepdims=True)
    t = (t - mt) * jax.lax.rsqrt(vt + 1e-5) * lnt_g_ref[...] + lnt_b_ref[...]
    t_out_ref[...] = jnp.maximum(t, 0.0)


@jax.jit
def _run(x, W1, b1, ln1_g, ln1_b, W2, b2, W3, b3, Wg1, bg1, Wg2, bg2,
         Wt, bt, lnt_g, lnt_b):
    keys = pl.pallas_call(
        _score_body,
        out_shape=jax.ShapeDtypeStruct((B, IN_DIM), jnp.int32),
        compiler_params=pltpu.CompilerParams(vmem_limit_bytes=100 * 1024 * 1024),
    )(x, W1, b1, ln1_g, ln1_b, W2, b2, W3, b3, Wg1, bg1, Wg2, bg2)

    th = _thresh_sc_build()(keys)

    transformed, mask = pl.pallas_call(
        _transform_body,
        out_shape=(
            jax.ShapeDtypeStruct((B, HID), jnp.float32),
            jax.ShapeDtypeStruct((B, IN_DIM), jnp.float32),
        ),
        compiler_params=pltpu.CompilerParams(vmem_limit_bytes=100 * 1024 * 1024),
    )(x, keys, th, Wt, bt, lnt_g, lnt_b)
    return transformed, mask


def kernel(x, W1, b1, ln1_g, ln1_b, W2, b2, W3, b3, Wg1, bg1, Wg2, bg2, Wt, bt, lnt_g, lnt_b):
    return _run(
        x, W1, b1.reshape(1, -1), ln1_g.reshape(1, -1), ln1_b.reshape(1, -1),
        W2, b2.reshape(1, -1), W3, b3.reshape(1, -1),
        Wg1, bg1.reshape(1, -1), Wg2, bg2.reshape(1, -1),
        Wt, bt.reshape(1, -1), lnt_g.reshape(1, -1), lnt_b.reshape(1, -1),
    )
